# Initial kernel scaffold; baseline (speedup 1.0000x reference)
import jax
import jax.numpy as jnp
from jax.experimental import pallas as pl


def kernel(f, edge_index, edge_length, edge_sh, edge_length_embedded, Wq, fck_W1, fck_W2, fcv_W1, fcv_W2, dot_W):
    raise NotImplementedError("write your pallas kernel here")


# speedup vs baseline: 2.0917x; 2.0917x over previous
import functools

import jax
import jax.numpy as jnp
import numpy as np
from jax import lax
from jax.experimental import pallas as pl
from jax.experimental.pallas import tpu as pltpu
from jax.experimental.pallas import tpu_sc as plsc

_MAX_RADIUS = 1.3

_NC = 2
_NS = 16
_NW = _NC * _NS
_CH = 128


def _silu_c():
    z = np.linspace(-12.0, 12.0, 200001)
    phi = np.exp(-0.5 * z * z) / np.sqrt(2.0 * np.pi)
    s = z / (1.0 + np.exp(-z))
    return float(1.0 / np.sqrt(np.trapz(s * s * phi, z)))


_SILU_C = _silu_c()


def _table_body(f_ref, wq_ref, dw_ref, o_ref):
    f = f_ref[...]
    q = jnp.dot(f, wq_ref[...], preferred_element_type=jnp.float32)
    qd = jnp.dot(q, dw_ref[...], preferred_element_type=jnp.float32) * 0.25
    o_ref[...] = jnp.concatenate([f, qd], axis=1)


def _make_table(f, Wq, dot_W, interpret=False):
    n, mul = f.shape
    return pl.pallas_call(
        _table_body,
        out_shape=jax.ShapeDtypeStruct((n, 2 * mul), jnp.float32),
        interpret=interpret,
    )(f, Wq, dot_W)


def _edge_body(nedges, ele_ref, td_ref, fs_ref, sh_ref, len_ref,
               w1k_ref, w2k_ref, w1v_ref, w2v_ref, o_ref):
    blk = ele_ref.shape[0]
    mul = fs_ref.shape[1]
    nb = ele_ref.shape[1]
    ele = ele_ref[...]
    hk = jax.nn.silu(jnp.dot(ele, w1k_ref[...],
                             preferred_element_type=jnp.float32)
                     * (1.0 / np.sqrt(nb))) * _SILU_C
    hv = jax.nn.silu(jnp.dot(ele, w1v_ref[...],
                             preferred_element_type=jnp.float32)
                     * (1.0 / np.sqrt(nb))) * _SILU_C
    fd = td_ref[:, :mul]
    qdd = td_ref[:, mul:]
    fs = fs_ref[...]
    k = jnp.zeros((blk, mul), jnp.float32)
    v = jnp.zeros((blk, mul), jnp.float32)
    for i in range(mul):
        wk_i = jnp.dot(hk, w2k_ref[:, i * mul:(i + 1) * mul],
                       preferred_element_type=jnp.float32)
        wv_i = jnp.dot(hv, w2v_ref[:, i * mul:(i + 1) * mul],
                       preferred_element_type=jnp.float32)
        k = k + fd[:, i:i + 1] * wk_i
        v = v + fs[:, i:i + 1] * wv_i
    sh = sh_ref[...]
    scale = sh * (1.0 / mul)
    k = k * scale
    v = v * scale
    logit = jnp.sum(qdd * k, axis=1, keepdims=True) * (1.0 / mul)
    x = 10.0 * (1.0 - len_ref[...] / _MAX_RADIUS)
    xs = jnp.where(x > 0.0, x, 1.0)
    cutoff = jnp.where(x > 0.0, jnp.exp(-1.0 / xs), 0.0)
    w = cutoff * jnp.exp(logit)
    eid = lax.broadcasted_iota(jnp.int32, (blk, 1), 0) + pl.program_id(0) * blk
    w = jnp.where(eid < nedges, w, 0.0)
    u = jnp.sqrt(w) * v
    o_ref[...] = jnp.concatenate([u, jnp.broadcast_to(w, (blk, mul))], axis=1)


def _edge_stage(ele_p, td, fs, sh_p, len_p, W1k, W2k, W1v, W2v, nedges,
                blk=2048, interpret=False):
    epad, nb = ele_p.shape
    mul = fs.shape[1]
    grid = (epad // blk,)
    return pl.pallas_call(
        functools.partial(_edge_body, nedges),
        grid=grid,
        in_specs=[
            pl.BlockSpec((blk, nb), lambda i: (i, 0)),
            pl.BlockSpec((blk, 2 * mul), lambda i: (i, 0)),
            pl.BlockSpec((blk, mul), lambda i: (i, 0)),
            pl.BlockSpec((blk, 1), lambda i: (i, 0)),
            pl.BlockSpec((blk, 1), lambda i: (i, 0)),
            pl.BlockSpec(W1k.shape, lambda i: (0, 0)),
            pl.BlockSpec(W2k.shape, lambda i: (0, 0)),
            pl.BlockSpec(W1v.shape, lambda i: (0, 0)),
            pl.BlockSpec(W2v.shape, lambda i: (0, 0)),
        ],
        out_specs=pl.BlockSpec((blk, 2 * mul), lambda i: (i, 0)),
        out_shape=jax.ShapeDtypeStruct((epad, 2 * mul), jnp.float32),
        interpret=interpret,
    )(ele_p, td, fs, sh_p, len_p, W1k, W2k, W1v, W2v)


def _norm_body(p0_ref, p1_ref, o_ref):
    mul = o_ref.shape[1]
    s = p0_ref[:, :mul] + p1_ref[:, :mul]
    z = p0_ref[:, mul:mul + 1] + p1_ref[:, mul:mul + 1]
    z = jnp.where(z == 0.0, 1.0, z)
    o_ref[...] = s * lax.rsqrt(z)


def _normalize(p0, p1, interpret=False):
    n = p0.shape[0]
    mul = p0.shape[1] // 2
    return pl.pallas_call(
        _norm_body,
        out_shape=jax.ShapeDtypeStruct((n, mul), jnp.float32),
        interpret=interpret,
    )(p0, p1)


def _gather_calls(table, f, dst3, src3, epad):
    n, tw = table.shape
    mul = f.shape[1]
    nw, j, ch = dst3.shape
    per = j * ch
    mesh = plsc.VectorSubcoreMesh(core_axis_name="c", subcore_axis_name="s")

    @functools.partial(
        pl.kernel, mesh=mesh,
        compiler_params=pltpu.CompilerParams(use_tc_tiling_on_sc=False),
        out_type=(jax.ShapeDtypeStruct((epad, tw), jnp.float32),
                  jax.ShapeDtypeStruct((epad, mul), jnp.float32)),
        scratch_types=[
            pltpu.VMEM((j, ch), jnp.int32),
            pltpu.VMEM((j, ch), jnp.int32),
            pltpu.VMEM((ch, tw), jnp.float32),
            pltpu.VMEM((ch, mul), jnp.float32),
        ],
    )
    def gk(t_hbm, f_hbm, d_hbm, s_hbm, td_out, fs_out, idx_d, idx_s, bt, bf):
        wid = lax.axis_index("s") * _NC + lax.axis_index("c")
        pltpu.sync_copy(d_hbm.at[wid], idx_d)
        pltpu.sync_copy(s_hbm.at[wid], idx_s)

        def body(jj, _):
            base = pl.multiple_of(wid * per + jj * ch, ch)
            pltpu.sync_copy(t_hbm.at[idx_d.at[jj]], bt)
            pltpu.sync_copy(bt, td_out.at[pl.ds(base, ch)])
            pltpu.sync_copy(f_hbm.at[idx_s.at[jj]], bf)
            pltpu.sync_copy(bf, fs_out.at[pl.ds(base, ch)])
            return 0

        lax.fori_loop(0, j, body, 0)

    return gk(table, f, dst3, src3)


def _scatter_call(rows, dst3, zeros, n):
    epad, tw = rows.shape
    nw, j, ch = dst3.shape
    per = j * ch
    mesh = plsc.VectorSubcoreMesh(core_axis_name="c", subcore_axis_name="s")

    @functools.partial(
        pl.kernel, mesh=mesh,
        compiler_params=pltpu.CompilerParams(use_tc_tiling_on_sc=False),
        out_type=jax.ShapeDtypeStruct((_NC, n, tw), jnp.float32),
        scratch_types=[
            pltpu.VMEM((j, ch), jnp.int32),
            pltpu.VMEM((ch, tw), jnp.float32),
            pltpu.VMEM_SHARED((n, tw), jnp.float32),
        ],
    )
    def sk(r_hbm, d_hbm, z_hbm, p_out, idx_v, buf, acc):
        cid = lax.axis_index("c")
        sid = lax.axis_index("s")
        wid = sid * _NC + cid
        pltpu.sync_copy(d_hbm.at[wid], idx_v)

        @pl.when(sid == 0)
        def _():
            pltpu.sync_copy(z_hbm, acc)

        plsc.subcore_barrier()

        def body(jj, _):
            base = pl.multiple_of(wid * per + jj * ch, ch)
            pltpu.sync_copy(r_hbm.at[pl.ds(base, ch)], buf)
            pltpu.sync_copy(buf, acc.at[idx_v.at[jj]], add=True)
            return 0

        lax.fori_loop(0, j, body, 0)
        plsc.subcore_barrier()

        @pl.when(sid == 0)
        def _():
            pltpu.sync_copy(acc, p_out.at[cid])

    return sk(rows, dst3, zeros)


def kernel(f, edge_index, edge_length, edge_sh, edge_length_embedded,
           Wq, fck_W1, fck_W2, fcv_W1, fcv_W2, dot_W):
    n, mul = f.shape
    e = edge_index.shape[1]
    nb = edge_length_embedded.shape[1]
    grain = _NW * _CH
    epad = ((e + grain - 1) // grain) * grain
    pad = epad - e
    per = epad // _NW
    j = per // _CH

    src = jnp.pad(edge_index[0], (0, pad)).reshape(_NW, j, _CH)
    dst = jnp.pad(edge_index[1], (0, pad)).reshape(_NW, j, _CH)
    ele_p = jnp.pad(edge_length_embedded, ((0, pad), (0, 0)))
    len_p = jnp.pad(edge_length, (0, pad)).reshape(epad, 1)
    sh_p = jnp.pad(edge_sh[:, 0], (0, pad)).reshape(epad, 1)

    table = _make_table(f, Wq, dot_W)
    td, fs = _gather_calls(table, f, dst, src, epad)
    rows = _edge_stage(ele_p, td, fs, sh_p, len_p,
                       fck_W1, fck_W2, fcv_W1, fcv_W2, e)
    parts = _scatter_call(rows, dst, jnp.zeros((n, 2 * mul), jnp.float32), n)
    return _normalize(parts[0], parts[1])


# speedup vs baseline: 5.7314x; 2.7401x over previous
import functools

import jax
import jax.numpy as jnp
import numpy as np
from jax import lax
from jax.experimental import pallas as pl
from jax.experimental.pallas import tpu as pltpu
from jax.experimental.pallas import tpu_sc as plsc

_MAX_RADIUS = 1.3

_NC = 2
_NS = 16
_NW = _NC * _NS
_CH = 128


def _silu_c():
    z = np.linspace(-12.0, 12.0, 200001)
    phi = np.exp(-0.5 * z * z) / np.sqrt(2.0 * np.pi)
    s = z / (1.0 + np.exp(-z))
    return float(1.0 / np.sqrt(np.trapz(s * s * phi, z)))


_SILU_C = _silu_c()


def _table_body(f_ref, wq_ref, dw_ref, o_ref):
    f = f_ref[...]
    q = jnp.dot(f, wq_ref[...], preferred_element_type=jnp.float32)
    qd = jnp.dot(q, dw_ref[...], preferred_element_type=jnp.float32) * 0.25
    o_ref[...] = jnp.concatenate([f, qd], axis=1)


def _make_table(f, Wq, dot_W, interpret=False):
    n, mul = f.shape
    return pl.pallas_call(
        _table_body,
        out_shape=jax.ShapeDtypeStruct((n, 2 * mul), jnp.float32),
        interpret=interpret,
    )(f, Wq, dot_W)


def _edge_body(nedges, mul, eleT_ref, tdT_ref, fsT_ref, shT_ref, lenT_ref,
               w1kT_ref, w1vT_ref, wc_ref, o_ref):
    blk = eleT_ref.shape[1]
    nb = eleT_ref.shape[0]
    ele = eleT_ref[...]
    hk = jax.nn.silu(jnp.dot(w1kT_ref[...], ele,
                             preferred_element_type=jnp.float32)
                     * (1.0 / np.sqrt(nb))) * _SILU_C
    hv = jax.nn.silu(jnp.dot(w1vT_ref[...], ele,
                             preferred_element_type=jnp.float32)
                     * (1.0 / np.sqrt(nb))) * _SILU_C
    fd = tdT_ref[:mul, :]
    qdd = tdT_ref[mul:, :]
    fs = fsT_ref[...]
    a_k = jnp.repeat(hk, mul, axis=0) * jnp.tile(fd, (mul, 1))
    a_v = jnp.repeat(hv, mul, axis=0) * jnp.tile(fs, (mul, 1))
    kv = jnp.dot(wc_ref[...], jnp.concatenate([a_k, a_v], axis=0),
                 preferred_element_type=jnp.float32)
    kv = kv * (shT_ref[...] * (1.0 / mul))
    k = kv[:mul, :]
    v = kv[mul:, :]
    logit = jnp.sum(qdd * k, axis=0, keepdims=True) * (1.0 / mul)
    x = 10.0 * (1.0 - lenT_ref[...] / _MAX_RADIUS)
    xs = jnp.where(x > 0.0, x, 1.0)
    cutoff = jnp.where(x > 0.0, jnp.exp(-1.0 / xs), 0.0)
    w = cutoff * jnp.exp(logit)
    eid = lax.broadcasted_iota(jnp.int32, (1, blk), 1) + pl.program_id(0) * blk
    w = jnp.where(eid < nedges, w, 0.0)
    u = jnp.sqrt(w) * v
    o_ref[...] = jnp.concatenate([u, jnp.broadcast_to(w, (mul, blk))], axis=0)


def _edge_stage(eleT, tdT, fsT, shT, lenT, W1k, W2k, W1v, W2v, nedges,
                blk=4096, interpret=False):
    nb, epad = eleT.shape
    mul = fsT.shape[0]
    hid = W1k.shape[1]
    w2kT = jnp.transpose(W2k.reshape(hid, mul, mul), (2, 0, 1)).reshape(mul, hid * mul)
    w2vT = jnp.transpose(W2v.reshape(hid, mul, mul), (2, 0, 1)).reshape(mul, hid * mul)
    zero = jnp.zeros((mul, hid * mul), jnp.float32)
    wc = jnp.concatenate([
        jnp.concatenate([w2kT, zero], axis=1),
        jnp.concatenate([zero, w2vT], axis=1),
    ], axis=0)
    grid = (epad // blk,)
    return pl.pallas_call(
        functools.partial(_edge_body, nedges, mul),
        grid=grid,
        in_specs=[
            pl.BlockSpec((nb, blk), lambda i: (0, i)),
            pl.BlockSpec((2 * mul, blk), lambda i: (0, i)),
            pl.BlockSpec((mul, blk), lambda i: (0, i)),
            pl.BlockSpec((1, blk), lambda i: (0, i)),
            pl.BlockSpec((1, blk), lambda i: (0, i)),
            pl.BlockSpec((mul, nb), lambda i: (0, 0)),
            pl.BlockSpec((mul, nb), lambda i: (0, 0)),
            pl.BlockSpec((2 * mul, 2 * hid * mul), lambda i: (0, 0)),
        ],
        out_specs=pl.BlockSpec((2 * mul, blk), lambda i: (0, i)),
        out_shape=jax.ShapeDtypeStruct((2 * mul, epad), jnp.float32),
        interpret=interpret,
    )(eleT, tdT, fsT, shT, lenT, W1k.T, W1v.T, wc)


def _norm_body(p0_ref, p1_ref, o_ref):
    mul = o_ref.shape[1]
    s = p0_ref[:, :mul] + p1_ref[:, :mul]
    z = p0_ref[:, mul:mul + 1] + p1_ref[:, mul:mul + 1]
    z = jnp.where(z == 0.0, 1.0, z)
    o_ref[...] = s * lax.rsqrt(z)


def _normalize(p0, p1, interpret=False):
    n = p0.shape[0]
    mul = p0.shape[1] // 2
    return pl.pallas_call(
        _norm_body,
        out_shape=jax.ShapeDtypeStruct((n, mul), jnp.float32),
        interpret=interpret,
    )(p0, p1)


def _gather_calls(table, f, dst3, src3, epad):
    n, tw = table.shape
    mul = f.shape[1]
    nw, j, ch = dst3.shape
    per = j * ch
    mesh = plsc.VectorSubcoreMesh(core_axis_name="c", subcore_axis_name="s")

    @functools.partial(
        pl.kernel, mesh=mesh,
        compiler_params=pltpu.CompilerParams(use_tc_tiling_on_sc=False),
        out_type=(jax.ShapeDtypeStruct((epad, tw), jnp.float32),
                  jax.ShapeDtypeStruct((epad, mul), jnp.float32)),
        scratch_types=[
            pltpu.VMEM((j, ch), jnp.int32),
            pltpu.VMEM((j, ch), jnp.int32),
            pltpu.VMEM((ch, tw), jnp.float32),
            pltpu.VMEM((ch, mul), jnp.float32),
        ],
    )
    def gk(t_hbm, f_hbm, d_hbm, s_hbm, td_out, fs_out, idx_d, idx_s, bt, bf):
        wid = lax.axis_index("s") * _NC + lax.axis_index("c")
        pltpu.sync_copy(d_hbm.at[wid], idx_d)
        pltpu.sync_copy(s_hbm.at[wid], idx_s)

        def body(jj, _):
            base = pl.multiple_of(wid * per + jj * ch, ch)
            pltpu.sync_copy(t_hbm.at[idx_d.at[jj]], bt)
            pltpu.sync_copy(bt, td_out.at[pl.ds(base, ch)])
            pltpu.sync_copy(f_hbm.at[idx_s.at[jj]], bf)
            pltpu.sync_copy(bf, fs_out.at[pl.ds(base, ch)])
            return 0

        lax.fori_loop(0, j, body, 0)

    return gk(table, f, dst3, src3)


def _scatter_call(rows, dst3, zeros, n):
    epad, tw = rows.shape
    nw, j, ch = dst3.shape
    per = j * ch
    mesh = plsc.VectorSubcoreMesh(core_axis_name="c", subcore_axis_name="s")

    @functools.partial(
        pl.kernel, mesh=mesh,
        compiler_params=pltpu.CompilerParams(use_tc_tiling_on_sc=False),
        out_type=jax.ShapeDtypeStruct((_NC, n, tw), jnp.float32),
        scratch_types=[
            pltpu.VMEM((j, ch), jnp.int32),
            pltpu.VMEM((ch, tw), jnp.float32),
            pltpu.VMEM_SHARED((n, tw), jnp.float32),
        ],
    )
    def sk(r_hbm, d_hbm, z_hbm, p_out, idx_v, buf, acc):
        cid = lax.axis_index("c")
        sid = lax.axis_index("s")
        wid = sid * _NC + cid
        pltpu.sync_copy(d_hbm.at[wid], idx_v)

        @pl.when(sid == 0)
        def _():
            pltpu.sync_copy(z_hbm, acc)

        plsc.subcore_barrier()

        def body(jj, _):
            base = pl.multiple_of(wid * per + jj * ch, ch)
            pltpu.sync_copy(r_hbm.at[pl.ds(base, ch)], buf)
            pltpu.sync_copy(buf, acc.at[idx_v.at[jj]], add=True)
            return 0

        lax.fori_loop(0, j, body, 0)
        plsc.subcore_barrier()

        @pl.when(sid == 0)
        def _():
            pltpu.sync_copy(acc, p_out.at[cid])

    return sk(rows, dst3, zeros)


def kernel(f, edge_index, edge_length, edge_sh, edge_length_embedded,
           Wq, fck_W1, fck_W2, fcv_W1, fcv_W2, dot_W):
    n, mul = f.shape
    e = edge_index.shape[1]
    nb = edge_length_embedded.shape[1]
    grain = _NW * _CH
    epad = ((e + grain - 1) // grain) * grain
    pad = epad - e
    per = epad // _NW
    j = per // _CH

    src = jnp.pad(edge_index[0], (0, pad)).reshape(_NW, j, _CH)
    dst = jnp.pad(edge_index[1], (0, pad)).reshape(_NW, j, _CH)
    eleT = jnp.pad(edge_length_embedded, ((0, pad), (0, 0))).T
    lenT = jnp.pad(edge_length, (0, pad)).reshape(1, epad)
    shT = jnp.pad(edge_sh[:, 0], (0, pad)).reshape(1, epad)

    table = _make_table(f, Wq, dot_W)
    td, fs = _gather_calls(table, f, dst, src, epad)
    outT = _edge_stage(eleT, td.T, fs.T, shT, lenT,
                       fck_W1, fck_W2, fcv_W1, fcv_W2, e)
    parts = _scatter_call(outT.T, dst, jnp.zeros((n, 2 * mul), jnp.float32), n)
    return _normalize(parts[0], parts[1])


# speedup vs baseline: 6.1446x; 1.0721x over previous
import functools

import jax
import jax.numpy as jnp
import numpy as np
from jax import lax
from jax.experimental import pallas as pl
from jax.experimental.pallas import tpu as pltpu
from jax.experimental.pallas import tpu_sc as plsc

_MAX_RADIUS = 1.3

_NC = 2
_NS = 16
_NW = _NC * _NS
_CH = 128


def _silu_c():
    z = np.linspace(-12.0, 12.0, 200001)
    phi = np.exp(-0.5 * z * z) / np.sqrt(2.0 * np.pi)
    s = z / (1.0 + np.exp(-z))
    return float(1.0 / np.sqrt(np.trapz(s * s * phi, z)))


_SILU_C = _silu_c()


def _table_body(f_ref, wq_ref, dw_ref, o_ref):
    f = f_ref[...]
    q = jnp.dot(f, wq_ref[...], preferred_element_type=jnp.float32)
    qd = jnp.dot(q, dw_ref[...], preferred_element_type=jnp.float32) * 0.25
    o_ref[...] = jnp.concatenate([f, qd], axis=1)


def _make_table(f, Wq, dot_W, interpret=False):
    n, mul = f.shape
    return pl.pallas_call(
        _table_body,
        out_shape=jax.ShapeDtypeStruct((n, 2 * mul), jnp.float32),
        interpret=interpret,
    )(f, Wq, dot_W)


def _edge_body(nedges, mul, eleT_ref, td_ref, fs_ref, shT_ref, lenT_ref,
               w1kT_ref, w1vT_ref, wc_ref, o_ref):
    blk = eleT_ref.shape[1]
    nb = eleT_ref.shape[0]
    ele = eleT_ref[...]
    hk = jax.nn.silu(jnp.dot(w1kT_ref[...], ele,
                             preferred_element_type=jnp.float32)
                     * (1.0 / np.sqrt(nb))) * _SILU_C
    hv = jax.nn.silu(jnp.dot(w1vT_ref[...], ele,
                             preferred_element_type=jnp.float32)
                     * (1.0 / np.sqrt(nb))) * _SILU_C
    tdT = td_ref[...].T
    fd = tdT[:mul, :]
    qdd = tdT[mul:, :]
    fs = fs_ref[...].T
    a_k = jnp.repeat(hk, mul, axis=0) * jnp.tile(fd, (mul, 1))
    a_v = jnp.repeat(hv, mul, axis=0) * jnp.tile(fs, (mul, 1))
    kv = jnp.dot(wc_ref[...], jnp.concatenate([a_k, a_v], axis=0),
                 preferred_element_type=jnp.float32)
    kv = kv * (shT_ref[...] * (1.0 / mul))
    k = kv[:mul, :]
    v = kv[mul:, :]
    logit = jnp.sum(qdd * k, axis=0, keepdims=True) * (1.0 / mul)
    x = 10.0 * (1.0 - lenT_ref[...] / _MAX_RADIUS)
    xs = jnp.where(x > 0.0, x, 1.0)
    cutoff = jnp.where(x > 0.0, jnp.exp(-1.0 / xs), 0.0)
    w = cutoff * jnp.exp(logit)
    eid = lax.broadcasted_iota(jnp.int32, (1, blk), 1) + pl.program_id(0) * blk
    w = jnp.where(eid < nedges, w, 0.0)
    u = jnp.sqrt(w) * v
    o_ref[...] = jnp.concatenate([u, jnp.broadcast_to(w, (mul, blk))], axis=0).T


def _edge_stage(eleT, td, fs, shT, lenT, W1k, W2k, W1v, W2v, nedges,
                blk=4096, interpret=False):
    nb, epad = eleT.shape
    mul = fs.shape[1]
    hid = W1k.shape[1]
    w2kT = jnp.transpose(W2k.reshape(hid, mul, mul), (2, 0, 1)).reshape(mul, hid * mul)
    w2vT = jnp.transpose(W2v.reshape(hid, mul, mul), (2, 0, 1)).reshape(mul, hid * mul)
    zero = jnp.zeros((mul, hid * mul), jnp.float32)
    wc = jnp.concatenate([
        jnp.concatenate([w2kT, zero], axis=1),
        jnp.concatenate([zero, w2vT], axis=1),
    ], axis=0)
    grid = (epad // blk,)
    return pl.pallas_call(
        functools.partial(_edge_body, nedges, mul),
        grid=grid,
        in_specs=[
            pl.BlockSpec((nb, blk), lambda i: (0, i)),
            pl.BlockSpec((blk, 2 * mul), lambda i: (i, 0)),
            pl.BlockSpec((blk, mul), lambda i: (i, 0)),
            pl.BlockSpec((1, blk), lambda i: (0, i)),
            pl.BlockSpec((1, blk), lambda i: (0, i)),
            pl.BlockSpec((mul, nb), lambda i: (0, 0)),
            pl.BlockSpec((mul, nb), lambda i: (0, 0)),
            pl.BlockSpec((2 * mul, 2 * hid * mul), lambda i: (0, 0)),
        ],
        out_specs=pl.BlockSpec((blk, 2 * mul), lambda i: (i, 0)),
        out_shape=jax.ShapeDtypeStruct((epad, 2 * mul), jnp.float32),
        interpret=interpret,
    )(eleT, td, fs, shT, lenT, W1k.T, W1v.T, wc)


def _norm_body(p0_ref, p1_ref, o_ref):
    mul = o_ref.shape[1]
    s = p0_ref[:, :mul] + p1_ref[:, :mul]
    z = p0_ref[:, mul:mul + 1] + p1_ref[:, mul:mul + 1]
    z = jnp.where(z == 0.0, 1.0, z)
    o_ref[...] = s * lax.rsqrt(z)


def _normalize(p0, p1, interpret=False):
    n = p0.shape[0]
    mul = p0.shape[1] // 2
    return pl.pallas_call(
        _norm_body,
        out_shape=jax.ShapeDtypeStruct((n, mul), jnp.float32),
        interpret=interpret,
    )(p0, p1)


def _gather_calls(table, f, dst3, src3, epad):
    n, tw = table.shape
    mul = f.shape[1]
    nw, j, ch = dst3.shape
    per = j * ch
    mesh = plsc.VectorSubcoreMesh(core_axis_name="c", subcore_axis_name="s")

    @functools.partial(
        pl.kernel, mesh=mesh,
        compiler_params=pltpu.CompilerParams(use_tc_tiling_on_sc=False),
        out_type=(jax.ShapeDtypeStruct((epad, tw), jnp.float32),
                  jax.ShapeDtypeStruct((epad, mul), jnp.float32)),
        scratch_types=[
            pltpu.VMEM((j, ch), jnp.int32),
            pltpu.VMEM((j, ch), jnp.int32),
            pltpu.VMEM((ch, tw), jnp.float32),
            pltpu.VMEM((ch, mul), jnp.float32),
        ],
    )
    def gk(t_hbm, f_hbm, d_hbm, s_hbm, td_out, fs_out, idx_d, idx_s, bt, bf):
        wid = lax.axis_index("s") * _NC + lax.axis_index("c")
        pltpu.sync_copy(d_hbm.at[wid], idx_d)
        pltpu.sync_copy(s_hbm.at[wid], idx_s)

        def body(jj, _):
            base = pl.multiple_of(wid * per + jj * ch, ch)
            pltpu.sync_copy(t_hbm.at[idx_d.at[jj]], bt)
            pltpu.sync_copy(bt, td_out.at[pl.ds(base, ch)])
            pltpu.sync_copy(f_hbm.at[idx_s.at[jj]], bf)
            pltpu.sync_copy(bf, fs_out.at[pl.ds(base, ch)])
            return 0

        lax.fori_loop(0, j, body, 0)

    return gk(table, f, dst3, src3)


def _scatter_call(rows, dst3, zeros, n):
    epad, tw = rows.shape
    nw, j, ch = dst3.shape
    per = j * ch
    mesh = plsc.VectorSubcoreMesh(core_axis_name="c", subcore_axis_name="s")

    @functools.partial(
        pl.kernel, mesh=mesh,
        compiler_params=pltpu.CompilerParams(use_tc_tiling_on_sc=False),
        out_type=jax.ShapeDtypeStruct((_NC, n, tw), jnp.float32),
        scratch_types=[
            pltpu.VMEM((j, ch), jnp.int32),
            pltpu.VMEM((ch, tw), jnp.float32),
            pltpu.VMEM_SHARED((n, tw), jnp.float32),
        ],
    )
    def sk(r_hbm, d_hbm, z_hbm, p_out, idx_v, buf, acc):
        cid = lax.axis_index("c")
        sid = lax.axis_index("s")
        wid = sid * _NC + cid
        pltpu.sync_copy(d_hbm.at[wid], idx_v)

        @pl.when(sid == 0)
        def _():
            pltpu.sync_copy(z_hbm, acc)

        plsc.subcore_barrier()

        def body(jj, _):
            base = pl.multiple_of(wid * per + jj * ch, ch)
            pltpu.sync_copy(r_hbm.at[pl.ds(base, ch)], buf)
            pltpu.sync_copy(buf, acc.at[idx_v.at[jj]], add=True)
            return 0

        lax.fori_loop(0, j, body, 0)
        plsc.subcore_barrier()

        @pl.when(sid == 0)
        def _():
            pltpu.sync_copy(acc, p_out.at[cid])

    return sk(rows, dst3, zeros)


def kernel(f, edge_index, edge_length, edge_sh, edge_length_embedded,
           Wq, fck_W1, fck_W2, fcv_W1, fcv_W2, dot_W):
    n, mul = f.shape
    e = edge_index.shape[1]
    nb = edge_length_embedded.shape[1]
    grain = _NW * _CH
    epad = ((e + grain - 1) // grain) * grain
    pad = epad - e
    per = epad // _NW
    j = per // _CH

    src = jnp.pad(edge_index[0], (0, pad)).reshape(_NW, j, _CH)
    dst = jnp.pad(edge_index[1], (0, pad)).reshape(_NW, j, _CH)
    eleT = jnp.pad(edge_length_embedded, ((0, pad), (0, 0))).T
    lenT = jnp.pad(edge_length, (0, pad)).reshape(1, epad)
    shT = jnp.pad(edge_sh[:, 0], (0, pad)).reshape(1, epad)

    table = _make_table(f, Wq, dot_W)
    td, fs = _gather_calls(table, f, dst, src, epad)
    rows = _edge_stage(eleT, td, fs, shT, lenT,
                       fck_W1, fck_W2, fcv_W1, fcv_W2, e)
    parts = _scatter_call(rows, dst, jnp.zeros((n, 2 * mul), jnp.float32), n)
    return _normalize(parts[0], parts[1])


# speedup vs baseline: 7.3538x; 1.1968x over previous
import functools

import jax
import jax.numpy as jnp
import numpy as np
from jax import lax
from jax.experimental import pallas as pl
from jax.experimental.pallas import tpu as pltpu
from jax.experimental.pallas import tpu_sc as plsc

_MAX_RADIUS = 1.3

_NC = 2
_NS = 16
_NW = _NC * _NS
_CH = 128


def _silu_c():
    z = np.linspace(-12.0, 12.0, 200001)
    phi = np.exp(-0.5 * z * z) / np.sqrt(2.0 * np.pi)
    s = z / (1.0 + np.exp(-z))
    return float(1.0 / np.sqrt(np.trapz(s * s * phi, z)))


_SILU_C = _silu_c()


def _table_body(f_ref, wq_ref, dw_ref, o_ref):
    f = f_ref[...]
    q = jnp.dot(f, wq_ref[...], preferred_element_type=jnp.float32)
    qd = jnp.dot(q, dw_ref[...], preferred_element_type=jnp.float32) * 0.25
    o_ref[...] = jnp.concatenate([f, qd], axis=1)


def _make_table(f, Wq, dot_W, interpret=False):
    n, mul = f.shape
    return pl.pallas_call(
        _table_body,
        out_shape=jax.ShapeDtypeStruct((n, 2 * mul), jnp.float32),
        interpret=interpret,
    )(f, Wq, dot_W)


def _edge_body(nedges, mul, eleT_ref, td_ref, fs_ref, shT_ref, lenT_ref,
               w1kT_ref, w1vT_ref, wc_ref, o_ref):
    blk = eleT_ref.shape[1]
    nb = eleT_ref.shape[0]
    ele = eleT_ref[...]
    hk = jax.nn.silu(jnp.dot(w1kT_ref[...], ele,
                             preferred_element_type=jnp.float32)
                     * (1.0 / np.sqrt(nb))) * _SILU_C
    hv = jax.nn.silu(jnp.dot(w1vT_ref[...], ele,
                             preferred_element_type=jnp.float32)
                     * (1.0 / np.sqrt(nb))) * _SILU_C
    tdT = td_ref[...].T
    fd = tdT[:mul, :]
    qdd = tdT[mul:, :]
    fs = fs_ref[...].T
    a_k = jnp.repeat(hk, mul, axis=0) * jnp.tile(fd, (mul, 1))
    a_v = jnp.repeat(hv, mul, axis=0) * jnp.tile(fs, (mul, 1))
    kv = jnp.dot(wc_ref[...], jnp.concatenate([a_k, a_v], axis=0),
                 preferred_element_type=jnp.float32)
    kv = kv * (shT_ref[...] * (1.0 / mul))
    k = kv[:mul, :]
    v = kv[mul:, :]
    logit = jnp.sum(qdd * k, axis=0, keepdims=True) * (1.0 / mul)
    x = 10.0 * (1.0 - lenT_ref[...] / _MAX_RADIUS)
    xs = jnp.where(x > 0.0, x, 1.0)
    cutoff = jnp.where(x > 0.0, jnp.exp(-1.0 / xs), 0.0)
    w = cutoff * jnp.exp(logit)
    eid = lax.broadcasted_iota(jnp.int32, (1, blk), 1) + pl.program_id(0) * blk
    w = jnp.where(eid < nedges, w, 0.0)
    u = jnp.sqrt(w) * v
    o_ref[...] = jnp.concatenate([u, jnp.broadcast_to(w, (mul, blk))], axis=0).T


def _edge_stage(eleT, td, fs, shT, lenT, W1k, W2k, W1v, W2v, nedges,
                blk=4096, interpret=False):
    nb, epad = eleT.shape
    mul = fs.shape[1]
    hid = W1k.shape[1]
    w2kT = jnp.transpose(W2k.reshape(hid, mul, mul), (2, 0, 1)).reshape(mul, hid * mul)
    w2vT = jnp.transpose(W2v.reshape(hid, mul, mul), (2, 0, 1)).reshape(mul, hid * mul)
    zero = jnp.zeros((mul, hid * mul), jnp.float32)
    wc = jnp.concatenate([
        jnp.concatenate([w2kT, zero], axis=1),
        jnp.concatenate([zero, w2vT], axis=1),
    ], axis=0)
    grid = (epad // blk,)
    return pl.pallas_call(
        functools.partial(_edge_body, nedges, mul),
        grid=grid,
        in_specs=[
            pl.BlockSpec((nb, blk), lambda i: (0, i)),
            pl.BlockSpec((blk, 2 * mul), lambda i: (i, 0)),
            pl.BlockSpec((blk, mul), lambda i: (i, 0)),
            pl.BlockSpec((1, blk), lambda i: (0, i)),
            pl.BlockSpec((1, blk), lambda i: (0, i)),
            pl.BlockSpec((mul, nb), lambda i: (0, 0)),
            pl.BlockSpec((mul, nb), lambda i: (0, 0)),
            pl.BlockSpec((2 * mul, 2 * hid * mul), lambda i: (0, 0)),
        ],
        out_specs=pl.BlockSpec((blk, 2 * mul), lambda i: (i, 0)),
        out_shape=jax.ShapeDtypeStruct((epad, 2 * mul), jnp.float32),
        interpret=interpret,
    )(eleT, td, fs, shT, lenT, W1k.T, W1v.T, wc)


def _norm_body(p0_ref, p1_ref, o_ref):
    mul = o_ref.shape[1]
    s = p0_ref[:, :mul] + p1_ref[:, :mul]
    z = p0_ref[:, mul:mul + 1] + p1_ref[:, mul:mul + 1]
    z = jnp.where(z == 0.0, 1.0, z)
    o_ref[...] = s * lax.rsqrt(z)


def _normalize(p0, p1, interpret=False):
    n = p0.shape[0]
    mul = p0.shape[1] // 2
    return pl.pallas_call(
        _norm_body,
        out_shape=jax.ShapeDtypeStruct((n, mul), jnp.float32),
        interpret=interpret,
    )(p0, p1)


def _gather_calls(table, f, dst3, src3, epad):
    n, tw = table.shape
    mul = f.shape[1]
    nw, j, ch = dst3.shape
    per = j * ch
    mesh = plsc.VectorSubcoreMesh(core_axis_name="c", subcore_axis_name="s")

    cpg = 8 if j % 8 == 0 else (10 if j % 10 == 0 else (5 if j % 5 == 0 else 1))
    ng = j // cpg
    rows_g = cpg * ch

    @functools.partial(
        pl.kernel, mesh=mesh,
        compiler_params=pltpu.CompilerParams(use_tc_tiling_on_sc=False),
        out_type=(jax.ShapeDtypeStruct((epad, tw), jnp.float32),
                  jax.ShapeDtypeStruct((epad, mul), jnp.float32)),
        scratch_types=[
            pltpu.VMEM((j, ch), jnp.int32),
            pltpu.VMEM((j, ch), jnp.int32),
            pltpu.VMEM((2 * rows_g, tw), jnp.float32),
            pltpu.VMEM((2 * rows_g, mul), jnp.float32),
            pltpu.SemaphoreType.DMA,
            pltpu.SemaphoreType.DMA,
        ],
    )
    def gk(t_hbm, f_hbm, d_hbm, s_hbm, td_out, fs_out,
           idx_d, idx_s, bt, bf, sg, sw):
        wid = lax.axis_index("s") * _NC + lax.axis_index("c")
        pltpu.sync_copy(d_hbm.at[wid], idx_d)
        pltpu.sync_copy(s_hbm.at[wid], idx_s)

        def issue_gets(g, b):
            gets = []
            for c in range(cpg):
                jj = g * cpg + c
                off = b * rows_g + c * ch
                gets.append(pltpu.async_copy(
                    t_hbm.at[idx_d.at[jj]], bt.at[pl.ds(off, ch)], sg))
                gets.append(pltpu.async_copy(
                    f_hbm.at[idx_s.at[jj]], bf.at[pl.ds(off, ch)], sg))
            return gets

        gets = {0: issue_gets(0, 0), 1: []}
        writes = {0: [], 1: []}
        for g in range(ng):
            b = g % 2
            o = (g + 1) % 2
            if g + 1 < ng:
                for h in writes[o]:
                    h.wait()
                gets[o] = issue_gets(g + 1, o)
            for h in gets[b]:
                h.wait()
            base = wid * per + g * rows_g
            writes[b] = [
                pltpu.async_copy(bt.at[pl.ds(b * rows_g, rows_g)],
                                 td_out.at[pl.ds(base, rows_g)], sw),
                pltpu.async_copy(bf.at[pl.ds(b * rows_g, rows_g)],
                                 fs_out.at[pl.ds(base, rows_g)], sw),
            ]
        for b in (0, 1):
            for h in writes[b]:
                h.wait()

    return gk(table, f, dst3, src3)


def _scatter_call(rows, dst3, zeros, n):
    epad, tw = rows.shape
    nw, j, ch = dst3.shape
    per = j * ch
    mesh = plsc.VectorSubcoreMesh(core_axis_name="c", subcore_axis_name="s")

    cpg = 8 if j % 8 == 0 else (10 if j % 10 == 0 else (5 if j % 5 == 0 else 1))
    ng = j // cpg
    rows_g = cpg * ch

    @functools.partial(
        pl.kernel, mesh=mesh,
        compiler_params=pltpu.CompilerParams(use_tc_tiling_on_sc=False),
        out_type=jax.ShapeDtypeStruct((_NC, n, tw), jnp.float32),
        scratch_types=[
            pltpu.VMEM((j, ch), jnp.int32),
            pltpu.VMEM((2 * rows_g, tw), jnp.float32),
            pltpu.VMEM_SHARED((n, tw), jnp.float32),
            pltpu.SemaphoreType.DMA,
            pltpu.SemaphoreType.DMA,
        ],
    )
    def sk(r_hbm, d_hbm, z_hbm, p_out, idx_v, buf, acc, sr, ss):
        cid = lax.axis_index("c")
        sid = lax.axis_index("s")
        wid = sid * _NC + cid
        pltpu.sync_copy(d_hbm.at[wid], idx_v)

        @pl.when(sid == 0)
        def _():
            pltpu.sync_copy(z_hbm, acc)

        plsc.subcore_barrier()

        def issue_read(g, b):
            base = wid * per + g * rows_g
            return [pltpu.async_copy(
                r_hbm.at[pl.ds(base, rows_g)],
                buf.at[pl.ds(b * rows_g, rows_g)], sr)]

        reads = {0: issue_read(0, 0), 1: []}
        scats = {0: [], 1: []}
        for g in range(ng):
            b = g % 2
            o = (g + 1) % 2
            if g + 1 < ng:
                for h in scats[o]:
                    h.wait()
                reads[o] = issue_read(g + 1, o)
            for h in reads[b]:
                h.wait()
            scats[b] = []
            for c in range(cpg):
                jj = g * cpg + c
                scats[b].append(pltpu.async_copy(
                    buf.at[pl.ds(b * rows_g + c * ch, ch)],
                    acc.at[idx_v.at[jj]], ss, add=True))
        for b in (0, 1):
            for h in scats[b]:
                h.wait()
        plsc.subcore_barrier()

        @pl.when(sid == 0)
        def _():
            pltpu.sync_copy(acc, p_out.at[cid])

    return sk(rows, dst3, zeros)


def kernel(f, edge_index, edge_length, edge_sh, edge_length_embedded,
           Wq, fck_W1, fck_W2, fcv_W1, fcv_W2, dot_W):
    n, mul = f.shape
    e = edge_index.shape[1]
    nb = edge_length_embedded.shape[1]
    grain = _NW * _CH
    epad = ((e + grain - 1) // grain) * grain
    pad = epad - e
    per = epad // _NW
    j = per // _CH

    src = jnp.pad(edge_index[0], (0, pad)).reshape(_NW, j, _CH)
    dst = jnp.pad(edge_index[1], (0, pad)).reshape(_NW, j, _CH)
    eleT = jnp.pad(edge_length_embedded, ((0, pad), (0, 0))).T
    lenT = jnp.pad(edge_length, (0, pad)).reshape(1, epad)
    shT = jnp.pad(edge_sh[:, 0], (0, pad)).reshape(1, epad)

    table = _make_table(f, Wq, dot_W)
    td, fs = _gather_calls(table, f, dst, src, epad)
    rows = _edge_stage(eleT, td, fs, shT, lenT,
                       fck_W1, fck_W2, fcv_W1, fcv_W2, e)
    parts = _scatter_call(rows, dst, jnp.zeros((n, 2 * mul), jnp.float32), n)
    return _normalize(parts[0], parts[1])
